# baseline (speedup 1.0000x reference)
import functools

import jax
import jax.numpy as jnp
from jax.experimental import pallas as pl
from jax.experimental.pallas import tpu as pltpu

BM = 64


def _moe_body(x_ref, wg_ref, bg_ref, wn_ref, bn_ref, noise_ref,
              weg_ref, weu_ref, wed_ref, wsg_ref, wsu_ref, wsd_ref,
              out_ref, gate_ref, pt_ref, mt_ref, xc_ref, y_ref, *, E, NH):
    e = pl.program_id(0)
    h = pl.program_id(1)
    xr = x_ref[...]
    T = xr.shape[0]

    @pl.when(jnp.logical_and(e == 0, h == 0))
    def _router():
        logits = jnp.dot(xr, wg_ref[...], preferred_element_type=jnp.float32) + bg_ref[...]
        nlog = jnp.dot(xr, wn_ref[...], preferred_element_type=jnp.float32) + bn_ref[...]
        sp = jnp.maximum(nlog, 0.0) + jnp.log1p(jnp.exp(-jnp.abs(nlog)))
        noisy = logits + noise_ref[...] * sp
        idx = jax.lax.broadcasted_iota(jnp.int32, noisy.shape, 1)
        m1 = jnp.max(noisy, axis=1, keepdims=True)
        am1 = jnp.min(jnp.where(noisy == m1, idx, E), axis=1, keepdims=True)
        n2 = jnp.where(idx == am1, -jnp.inf, noisy)
        m2 = jnp.max(n2, axis=1, keepdims=True)
        am2 = jnp.min(jnp.where(n2 == m2, idx, E), axis=1, keepdims=True)
        mask = jnp.logical_or(idx == am1, idx == am2)
        ex = jnp.where(mask, jnp.exp(noisy - m1), 0.0)
        gate_ref[...] = ex / jnp.sum(ex, axis=1, keepdims=True)
        maskf = mask.astype(jnp.float32)
        r_i = jax.lax.broadcasted_iota(jnp.int32, (T, T), 0)
        c_i = jax.lax.broadcasted_iota(jnp.int32, (T, T), 1)
        ident = (r_i == c_i).astype(jnp.float32)
        upper = (r_i <= c_i).astype(jnp.float32)
        mT = jax.lax.dot_general(maskf, ident, (((0,), (0,)), ((), ())),
                                 preferred_element_type=jnp.float32)
        mt_ref[...] = mT
        pt_ref[...] = jnp.dot(mT, upper, preferred_element_type=jnp.float32)
        out_ref[...] = jnp.zeros_like(out_ref)

    def mlp_rows(nb, wg, wu, wd):
        def lbody(rb, _):
            xs = xc_ref[pl.ds(rb * BM, BM), :]
            g = jnp.dot(xs, wg, preferred_element_type=jnp.float32)
            u = jnp.dot(xs, wu, preferred_element_type=jnp.float32)
            act = (g / (1.0 + jnp.exp(-g))) * u
            y_ref[pl.ds(rb * BM, BM), :] += jnp.dot(
                act, wd, preferred_element_type=jnp.float32)
            return 0
        jax.lax.fori_loop(0, nb, lbody, 0)

    @pl.when(e < E)
    def _routed():
        eor = (jax.lax.broadcasted_iota(jnp.int32, (1, E), 1) == e).astype(jnp.float32)
        prow = jnp.dot(eor, pt_ref[...], preferred_element_type=jnp.float32)
        mrow = jnp.dot(eor, mt_ref[...], preferred_element_type=jnp.float32)
        n_i = jnp.sum(mrow).astype(jnp.int32)
        nb = (n_i + BM - 1) // BM

        def sel_matrix():
            r_i = jax.lax.broadcasted_iota(jnp.int32, (T, T), 0).astype(jnp.float32)
            return jnp.where((prow - 1.0 == r_i) & (mrow > 0.0), 1.0, 0.0)

        @pl.when(h == 0)
        def _gather():
            s = sel_matrix()
            xc_ref[...] = jnp.dot(s, xr, preferred_element_type=jnp.float32)
            y_ref[...] = jnp.zeros_like(y_ref)

        mlp_rows(nb, weg_ref[0], weu_ref[0], wed_ref[0])

        @pl.when(h == NH - 1)
        def _scatter():
            s = sel_matrix()
            eoc = (jax.lax.broadcasted_iota(jnp.int32, (E, 1), 0) == e).astype(jnp.float32)
            gcol = jnp.dot(gate_ref[...], eoc, preferred_element_type=jnp.float32)
            gs = jnp.dot(s, gcol, preferred_element_type=jnp.float32)
            out_ref[...] += jax.lax.dot_general(
                s, y_ref[...] * gs, (((0,), (0,)), ((), ())),
                preferred_element_type=jnp.float32)

    @pl.when(e >= E)
    def _shared():
        g = jnp.dot(xr, wsg_ref[0], preferred_element_type=jnp.float32)
        u = jnp.dot(xr, wsu_ref[0], preferred_element_type=jnp.float32)
        act = (g / (1.0 + jnp.exp(-g))) * u
        out_ref[...] += jnp.dot(act, wsd_ref[0], preferred_element_type=jnp.float32)


def kernel(x, Wg, bg, Wn, bn, We_gate, We_up, We_down, Ws_gate, Ws_up, Ws_down):
    B, S, D = x.shape
    E, _, H = We_gate.shape
    NS = Ws_gate.shape[0]
    T = B * S
    xf = x.reshape(T, D)
    noise = jax.random.normal(jax.random.key(42), (B, S, E), jnp.float32).reshape(T, E)
    BH = 512
    NH = H // BH
    grid = (E + NS, NH)

    def we_map(e, h):
        return (jnp.minimum(e, E - 1), 0, jnp.where(e < E, h, NH - 1))

    def wed_map(e, h):
        return (jnp.minimum(e, E - 1), jnp.where(e < E, h, NH - 1), 0)

    def ws_map(e, h):
        return (jnp.maximum(e - E, 0), 0, jnp.where(e < E, 0, h))

    def wsd_map(e, h):
        return (jnp.maximum(e - E, 0), jnp.where(e < E, 0, h), 0)

    const2 = lambda e, h: (0, 0)
    body = functools.partial(_moe_body, E=E, NH=NH)
    out = pl.pallas_call(
        body,
        grid=grid,
        in_specs=[
            pl.BlockSpec((T, D), const2),
            pl.BlockSpec((D, E), const2),
            pl.BlockSpec((1, E), const2),
            pl.BlockSpec((D, E), const2),
            pl.BlockSpec((1, E), const2),
            pl.BlockSpec((T, E), const2),
            pl.BlockSpec((1, D, BH), we_map),
            pl.BlockSpec((1, D, BH), we_map),
            pl.BlockSpec((1, BH, D), wed_map),
            pl.BlockSpec((1, D, BH), ws_map),
            pl.BlockSpec((1, D, BH), ws_map),
            pl.BlockSpec((1, BH, D), wsd_map),
        ],
        out_specs=pl.BlockSpec((T, D), const2),
        out_shape=jax.ShapeDtypeStruct((T, D), jnp.float32),
        scratch_shapes=[
            pltpu.VMEM((T, E), jnp.float32),
            pltpu.VMEM((E, T), jnp.float32),
            pltpu.VMEM((E, T), jnp.float32),
            pltpu.VMEM((T, D), jnp.float32),
            pltpu.VMEM((T, D), jnp.float32),
        ],
        compiler_params=pltpu.CompilerParams(
            dimension_semantics=("arbitrary", "arbitrary")),
    )(xf, Wg, bg.reshape(1, E), Wn, bn.reshape(1, E), noise,
      We_gate, We_up, We_down, Ws_gate, Ws_up, Ws_down)
    return out.reshape(B, S, D)


# speedup vs baseline: 1.1209x; 1.1209x over previous
import functools

import jax
import jax.numpy as jnp
from jax.experimental import pallas as pl
from jax.experimental.pallas import tpu as pltpu

BM = 128


def _moe_body(x_ref, wg_ref, bg_ref, wn_ref, bn_ref, noise_ref,
              weg_ref, weu_ref, wed_ref, wsg_ref, wsu_ref, wsd_ref,
              out_ref, gate_ref, pt_ref, mt_ref, xc_ref, y_ref, *, E, NH):
    e = pl.program_id(0)
    h = pl.program_id(1)
    xr = x_ref[...]
    T = xr.shape[0]

    @pl.when(jnp.logical_and(e == 0, h == 0))
    def _router():
        logits = jnp.dot(xr, wg_ref[...], preferred_element_type=jnp.float32) + bg_ref[...]
        nlog = jnp.dot(xr, wn_ref[...], preferred_element_type=jnp.float32) + bn_ref[...]
        sp = jnp.maximum(nlog, 0.0) + jnp.log1p(jnp.exp(-jnp.abs(nlog)))
        noisy = logits + noise_ref[...] * sp
        idx = jax.lax.broadcasted_iota(jnp.int32, noisy.shape, 1)
        m1 = jnp.max(noisy, axis=1, keepdims=True)
        am1 = jnp.min(jnp.where(noisy == m1, idx, E), axis=1, keepdims=True)
        n2 = jnp.where(idx == am1, -jnp.inf, noisy)
        m2 = jnp.max(n2, axis=1, keepdims=True)
        am2 = jnp.min(jnp.where(n2 == m2, idx, E), axis=1, keepdims=True)
        mask = jnp.logical_or(idx == am1, idx == am2)
        ex = jnp.where(mask, jnp.exp(noisy - m1), 0.0)
        gate_ref[...] = ex / jnp.sum(ex, axis=1, keepdims=True)
        maskf = mask.astype(jnp.float32)
        r_i = jax.lax.broadcasted_iota(jnp.int32, (T, T), 0)
        c_i = jax.lax.broadcasted_iota(jnp.int32, (T, T), 1)
        ident = (r_i == c_i).astype(jnp.float32)
        upper = (r_i <= c_i).astype(jnp.float32)
        mT = jax.lax.dot_general(maskf, ident, (((0,), (0,)), ((), ())),
                                 preferred_element_type=jnp.float32)
        mt_ref[...] = mT
        pt_ref[...] = jnp.dot(mT, upper, preferred_element_type=jnp.float32)
        out_ref[...] = jnp.zeros_like(out_ref)

    @pl.when(e < E)
    def _routed():
        eor = (jax.lax.broadcasted_iota(jnp.int32, (1, E), 1) == e).astype(jnp.float32)
        prow = jnp.dot(eor, pt_ref[...], preferred_element_type=jnp.float32)
        mrow = jnp.dot(eor, mt_ref[...], preferred_element_type=jnp.float32)
        overflow = jnp.sum(mrow) > BM

        def sel_matrix():
            r_i = jax.lax.broadcasted_iota(jnp.int32, (T, T), 0).astype(jnp.float32)
            return jnp.where((prow - 1.0 == r_i) & (mrow > 0.0), 1.0, 0.0)

        def mlp_block(lo):
            xs = xc_ref[lo:lo + BM, :]
            g = jnp.dot(xs, weg_ref[0], preferred_element_type=jnp.float32)
            u = jnp.dot(xs, weu_ref[0], preferred_element_type=jnp.float32)
            act = (g / (1.0 + jnp.exp(-g))) * u
            val = jnp.dot(act, wed_ref[0], preferred_element_type=jnp.float32)
            y_ref[lo:lo + BM, :] = jnp.where(h == 0, val, y_ref[lo:lo + BM, :] + val)

        @pl.when(h == 0)
        def _gather():
            s = sel_matrix()
            xc_ref[0:BM, :] = jnp.dot(s[0:BM, :], xr, preferred_element_type=jnp.float32)

            @pl.when(overflow)
            def _():
                xc_ref[BM:, :] = jnp.dot(s[BM:, :], xr, preferred_element_type=jnp.float32)

        mlp_block(0)

        @pl.when(overflow)
        def _():
            mlp_block(BM)

        @pl.when(h == NH - 1)
        def _scatter():
            s = sel_matrix()
            eoc = (jax.lax.broadcasted_iota(jnp.int32, (E, 1), 0) == e).astype(jnp.float32)
            gcol = jnp.dot(gate_ref[...], eoc, preferred_element_type=jnp.float32)
            gs = jnp.dot(s, gcol, preferred_element_type=jnp.float32)
            out_ref[...] += jax.lax.dot_general(
                s[0:BM, :], y_ref[0:BM, :] * gs[0:BM, :], (((0,), (0,)), ((), ())),
                preferred_element_type=jnp.float32)

            @pl.when(overflow)
            def _():
                out_ref[...] += jax.lax.dot_general(
                    s[BM:, :], y_ref[BM:, :] * gs[BM:, :], (((0,), (0,)), ((), ())),
                    preferred_element_type=jnp.float32)

    @pl.when(e >= E)
    def _shared():
        g = jnp.dot(xr, wsg_ref[0], preferred_element_type=jnp.float32)
        u = jnp.dot(xr, wsu_ref[0], preferred_element_type=jnp.float32)
        act = (g / (1.0 + jnp.exp(-g))) * u
        out_ref[...] += jnp.dot(act, wsd_ref[0], preferred_element_type=jnp.float32)


def kernel(x, Wg, bg, Wn, bn, We_gate, We_up, We_down, Ws_gate, Ws_up, Ws_down):
    B, S, D = x.shape
    E, _, H = We_gate.shape
    NS = Ws_gate.shape[0]
    T = B * S
    xf = x.reshape(T, D)
    noise = jax.random.normal(jax.random.key(42), (B, S, E), jnp.float32).reshape(T, E)
    BH = 512
    NH = H // BH
    grid = (E + NS, NH)

    def we_map(e, h):
        return (jnp.minimum(e, E - 1), 0, jnp.where(e < E, h, NH - 1))

    def wed_map(e, h):
        return (jnp.minimum(e, E - 1), jnp.where(e < E, h, NH - 1), 0)

    def ws_map(e, h):
        return (jnp.maximum(e - E, 0), 0, jnp.where(e < E, 0, h))

    def wsd_map(e, h):
        return (jnp.maximum(e - E, 0), jnp.where(e < E, 0, h), 0)

    const2 = lambda e, h: (0, 0)
    body = functools.partial(_moe_body, E=E, NH=NH)
    out = pl.pallas_call(
        body,
        grid=grid,
        in_specs=[
            pl.BlockSpec((T, D), const2),
            pl.BlockSpec((D, E), const2),
            pl.BlockSpec((1, E), const2),
            pl.BlockSpec((D, E), const2),
            pl.BlockSpec((1, E), const2),
            pl.BlockSpec((T, E), const2),
            pl.BlockSpec((1, D, BH), we_map),
            pl.BlockSpec((1, D, BH), we_map),
            pl.BlockSpec((1, BH, D), wed_map),
            pl.BlockSpec((1, D, BH), ws_map),
            pl.BlockSpec((1, D, BH), ws_map),
            pl.BlockSpec((1, BH, D), wsd_map),
        ],
        out_specs=pl.BlockSpec((T, D), const2),
        out_shape=jax.ShapeDtypeStruct((T, D), jnp.float32),
        scratch_shapes=[
            pltpu.VMEM((T, E), jnp.float32),
            pltpu.VMEM((E, T), jnp.float32),
            pltpu.VMEM((E, T), jnp.float32),
            pltpu.VMEM((T, D), jnp.float32),
            pltpu.VMEM((T, D), jnp.float32),
        ],
        compiler_params=pltpu.CompilerParams(
            dimension_semantics=("arbitrary", "arbitrary")),
    )(xf, Wg, bg.reshape(1, E), Wn, bn.reshape(1, E), noise,
      We_gate, We_up, We_down, Ws_gate, Ws_up, Ws_down)
    return out.reshape(B, S, D)


# speedup vs baseline: 1.3097x; 1.1684x over previous
import functools

import jax
import jax.numpy as jnp
from jax.experimental import pallas as pl
from jax.experimental.pallas import tpu as pltpu

BM = 128


def _moe_body(x_ref, wg_ref, bg_ref, wn_ref, bn_ref, noise_ref,
              weg_ref, weu_ref, wed_ref, wsg_ref, wsu_ref, wsd_ref,
              out_ref, gate_ref, pt_ref, mt_ref, xc_ref, y_ref, *, E, NH):
    e = pl.program_id(0)
    h = pl.program_id(1)
    xr = x_ref[...]
    T = xr.shape[0]

    @pl.when(jnp.logical_and(e == 0, h == 0))
    def _router():
        logits = jnp.dot(xr, wg_ref[...], preferred_element_type=jnp.float32) + bg_ref[...]
        nlog = jnp.dot(xr, wn_ref[...], preferred_element_type=jnp.float32) + bn_ref[...]
        sp = jnp.maximum(nlog, 0.0) + jnp.log1p(jnp.exp(-jnp.abs(nlog)))
        noisy = logits + noise_ref[...] * sp
        idx = jax.lax.broadcasted_iota(jnp.int32, noisy.shape, 1)
        m1 = jnp.max(noisy, axis=1, keepdims=True)
        am1 = jnp.min(jnp.where(noisy == m1, idx, E), axis=1, keepdims=True)
        n2 = jnp.where(idx == am1, -jnp.inf, noisy)
        m2 = jnp.max(n2, axis=1, keepdims=True)
        am2 = jnp.min(jnp.where(n2 == m2, idx, E), axis=1, keepdims=True)
        mask = jnp.logical_or(idx == am1, idx == am2)
        ex = jnp.where(mask, jnp.exp(noisy - m1), 0.0)
        gate_ref[...] = ex / jnp.sum(ex, axis=1, keepdims=True)
        maskf = mask.astype(jnp.float32)
        r_i = jax.lax.broadcasted_iota(jnp.int32, (T, T), 0)
        c_i = jax.lax.broadcasted_iota(jnp.int32, (T, T), 1)
        ident = (r_i == c_i).astype(jnp.float32)
        upper = (r_i <= c_i).astype(jnp.float32)
        mT = jax.lax.dot_general(maskf, ident, (((0,), (0,)), ((), ())),
                                 preferred_element_type=jnp.float32)
        mt_ref[...] = mT
        pt_ref[...] = jnp.dot(mT, upper, preferred_element_type=jnp.float32)
        out_ref[...] = jnp.zeros_like(out_ref)

    @pl.when(e < E)
    def _routed():
        eor = (jax.lax.broadcasted_iota(jnp.int32, (1, E), 1) == e).astype(jnp.float32)
        prow = jnp.dot(eor, pt_ref[...], preferred_element_type=jnp.float32)
        mrow = jnp.dot(eor, mt_ref[...], preferred_element_type=jnp.float32)
        overflow = jnp.sum(mrow) > BM

        def sel_matrix():
            r_i = jax.lax.broadcasted_iota(jnp.int32, (T, T), 0).astype(jnp.float32)
            return jnp.where((prow - 1.0 == r_i) & (mrow > 0.0), 1.0, 0.0)

        def mlp_block(lo):
            xs = xc_ref[lo:lo + BM, :]
            g = jnp.dot(xs, weg_ref[0], preferred_element_type=jnp.float32)
            u = jnp.dot(xs, weu_ref[0], preferred_element_type=jnp.float32)
            act = (g / (1.0 + jnp.exp(-g))) * u
            val = jnp.dot(act, wed_ref[0], preferred_element_type=jnp.float32)
            y_ref[lo:lo + BM, :] = jnp.where(h == 0, val, y_ref[lo:lo + BM, :] + val)

        @pl.when(h == 0)
        def _gather():
            s = sel_matrix()
            xc_ref[0:BM, :] = jnp.dot(s[0:BM, :], xr, preferred_element_type=jnp.float32)

            @pl.when(overflow)
            def _():
                xc_ref[BM:, :] = jnp.dot(s[BM:, :], xr, preferred_element_type=jnp.float32)

        mlp_block(0)

        @pl.when(overflow)
        def _():
            mlp_block(BM)

        @pl.when(h == NH - 1)
        def _scatter():
            s = sel_matrix()
            eoc = (jax.lax.broadcasted_iota(jnp.int32, (E, 1), 0) == e).astype(jnp.float32)
            gcol = jnp.dot(gate_ref[...], eoc, preferred_element_type=jnp.float32)
            gs = jnp.dot(s, gcol, preferred_element_type=jnp.float32)
            out_ref[...] += jax.lax.dot_general(
                s[0:BM, :], y_ref[0:BM, :] * gs[0:BM, :], (((0,), (0,)), ((), ())),
                preferred_element_type=jnp.float32)

            @pl.when(overflow)
            def _():
                out_ref[...] += jax.lax.dot_general(
                    s[BM:, :], y_ref[BM:, :] * gs[BM:, :], (((0,), (0,)), ((), ())),
                    preferred_element_type=jnp.float32)

    @pl.when(e >= E)
    def _shared():
        g = jnp.dot(xr, wsg_ref[0], preferred_element_type=jnp.float32)
        u = jnp.dot(xr, wsu_ref[0], preferred_element_type=jnp.float32)
        act = (g / (1.0 + jnp.exp(-g))) * u
        out_ref[...] += jnp.dot(act, wsd_ref[0], preferred_element_type=jnp.float32)


def kernel(x, Wg, bg, Wn, bn, We_gate, We_up, We_down, Ws_gate, Ws_up, Ws_down):
    B, S, D = x.shape
    E, _, H = We_gate.shape
    NS = Ws_gate.shape[0]
    T = B * S
    xf = x.reshape(T, D)
    noise = jax.random.normal(jax.random.key(42), (B, S, E), jnp.float32).reshape(T, E)
    BH = 1024
    NH = H // BH
    grid = (E + NS, NH)

    def we_map(e, h):
        return (jnp.minimum(e, E - 1), 0, jnp.where(e < E, h, NH - 1))

    def wed_map(e, h):
        return (jnp.minimum(e, E - 1), jnp.where(e < E, h, NH - 1), 0)

    def ws_map(e, h):
        return (jnp.maximum(e - E, 0), 0, jnp.where(e < E, 0, h))

    def wsd_map(e, h):
        return (jnp.maximum(e - E, 0), jnp.where(e < E, 0, h), 0)

    const2 = lambda e, h: (0, 0)
    body = functools.partial(_moe_body, E=E, NH=NH)
    out = pl.pallas_call(
        body,
        grid=grid,
        in_specs=[
            pl.BlockSpec((T, D), const2),
            pl.BlockSpec((D, E), const2),
            pl.BlockSpec((1, E), const2),
            pl.BlockSpec((D, E), const2),
            pl.BlockSpec((1, E), const2),
            pl.BlockSpec((T, E), const2),
            pl.BlockSpec((1, D, BH), we_map),
            pl.BlockSpec((1, D, BH), we_map),
            pl.BlockSpec((1, BH, D), wed_map),
            pl.BlockSpec((1, D, BH), ws_map),
            pl.BlockSpec((1, D, BH), ws_map),
            pl.BlockSpec((1, BH, D), wsd_map),
        ],
        out_specs=pl.BlockSpec((T, D), const2),
        out_shape=jax.ShapeDtypeStruct((T, D), jnp.float32),
        scratch_shapes=[
            pltpu.VMEM((T, E), jnp.float32),
            pltpu.VMEM((E, T), jnp.float32),
            pltpu.VMEM((E, T), jnp.float32),
            pltpu.VMEM((T, D), jnp.float32),
            pltpu.VMEM((T, D), jnp.float32),
        ],
        compiler_params=pltpu.CompilerParams(
            dimension_semantics=("arbitrary", "arbitrary")),
    )(xf, Wg, bg.reshape(1, E), Wn, bn.reshape(1, E), noise,
      We_gate, We_up, We_down, Ws_gate, Ws_up, Ws_down)
    return out.reshape(B, S, D)


# speedup vs baseline: 1.3521x; 1.0324x over previous
import functools

import jax
import jax.numpy as jnp
from jax.experimental import pallas as pl
from jax.experimental.pallas import tpu as pltpu

BM = 128


def _moe_body(x_ref, wg_ref, bg_ref, wn_ref, bn_ref, noise_ref,
              weg_ref, weu_ref, wed_ref, wsg_ref, wsu_ref, wsd_ref,
              out_ref, gate_ref, pt_ref, mt_ref, xc_ref, y_ref, *, E, NH):
    e = pl.program_id(0)
    h = pl.program_id(1)
    xr = x_ref[...]
    T = xr.shape[0]

    @pl.when(jnp.logical_and(e == 0, h == 0))
    def _router():
        logits = jnp.dot(xr, wg_ref[...], preferred_element_type=jnp.float32) + bg_ref[...]
        nlog = jnp.dot(xr, wn_ref[...], preferred_element_type=jnp.float32) + bn_ref[...]
        sp = jnp.maximum(nlog, 0.0) + jnp.log1p(jnp.exp(-jnp.abs(nlog)))
        noisy = logits + noise_ref[...] * sp
        idx = jax.lax.broadcasted_iota(jnp.int32, noisy.shape, 1)
        m1 = jnp.max(noisy, axis=1, keepdims=True)
        am1 = jnp.min(jnp.where(noisy == m1, idx, E), axis=1, keepdims=True)
        n2 = jnp.where(idx == am1, -jnp.inf, noisy)
        m2 = jnp.max(n2, axis=1, keepdims=True)
        am2 = jnp.min(jnp.where(n2 == m2, idx, E), axis=1, keepdims=True)
        mask = jnp.logical_or(idx == am1, idx == am2)
        ex = jnp.where(mask, jnp.exp(noisy - m1), 0.0)
        gate_ref[...] = ex / jnp.sum(ex, axis=1, keepdims=True)
        maskf = mask.astype(jnp.float32)
        r_i = jax.lax.broadcasted_iota(jnp.int32, (T, T), 0)
        c_i = jax.lax.broadcasted_iota(jnp.int32, (T, T), 1)
        ident = (r_i == c_i).astype(jnp.float32)
        upper = (r_i <= c_i).astype(jnp.float32)
        mT = jax.lax.dot_general(maskf, ident, (((0,), (0,)), ((), ())),
                                 preferred_element_type=jnp.float32)
        mt_ref[...] = mT
        pt_ref[...] = jnp.dot(mT, upper, preferred_element_type=jnp.float32)
        out_ref[...] = jnp.zeros_like(out_ref)

    @pl.when(e < E)
    def _routed():
        eor = (jax.lax.broadcasted_iota(jnp.int32, (1, E), 1) == e).astype(jnp.float32)
        prow = jnp.dot(eor, pt_ref[...], preferred_element_type=jnp.float32)
        mrow = jnp.dot(eor, mt_ref[...], preferred_element_type=jnp.float32)
        overflow = jnp.sum(mrow) > BM

        def sel_matrix():
            r_i = jax.lax.broadcasted_iota(jnp.int32, (T, T), 0).astype(jnp.float32)
            return jnp.where((prow - 1.0 == r_i) & (mrow > 0.0), 1.0, 0.0)

        def mlp_block(lo):
            xs = xc_ref[lo:lo + BM, :]
            g = jnp.dot(xs, weg_ref[0], preferred_element_type=jnp.float32)
            u = jnp.dot(xs, weu_ref[0], preferred_element_type=jnp.float32)
            act = (g / (1.0 + jnp.exp(-g))) * u
            val = jnp.dot(act, wed_ref[0], preferred_element_type=jnp.float32)
            y_ref[lo:lo + BM, :] = jnp.where(h == 0, val, y_ref[lo:lo + BM, :] + val)

        @pl.when(h == 0)
        def _gather():
            s = sel_matrix()
            xc_ref[0:BM, :] = jnp.dot(s[0:BM, :], xr, preferred_element_type=jnp.float32)

            @pl.when(overflow)
            def _():
                xc_ref[BM:, :] = jnp.dot(s[BM:, :], xr, preferred_element_type=jnp.float32)

        mlp_block(0)

        @pl.when(overflow)
        def _():
            mlp_block(BM)

        @pl.when(h == NH - 1)
        def _scatter():
            s = sel_matrix()
            eoc = (jax.lax.broadcasted_iota(jnp.int32, (E, 1), 0) == e).astype(jnp.float32)
            gcol = jnp.dot(gate_ref[...], eoc, preferred_element_type=jnp.float32)
            gs = jnp.dot(s, gcol, preferred_element_type=jnp.float32)
            out_ref[...] += jax.lax.dot_general(
                s[0:BM, :], y_ref[0:BM, :] * gs[0:BM, :], (((0,), (0,)), ((), ())),
                preferred_element_type=jnp.float32)

            @pl.when(overflow)
            def _():
                out_ref[...] += jax.lax.dot_general(
                    s[BM:, :], y_ref[BM:, :] * gs[BM:, :], (((0,), (0,)), ((), ())),
                    preferred_element_type=jnp.float32)

    @pl.when(e >= E)
    def _shared():
        g = jnp.dot(xr, wsg_ref[0], preferred_element_type=jnp.float32)
        u = jnp.dot(xr, wsu_ref[0], preferred_element_type=jnp.float32)
        act = (g / (1.0 + jnp.exp(-g))) * u
        out_ref[...] += jnp.dot(act, wsd_ref[0], preferred_element_type=jnp.float32)


def kernel(x, Wg, bg, Wn, bn, We_gate, We_up, We_down, Ws_gate, Ws_up, Ws_down):
    B, S, D = x.shape
    E, _, H = We_gate.shape
    NS = Ws_gate.shape[0]
    T = B * S
    xf = x.reshape(T, D)
    noise = jax.random.normal(jax.random.key(42), (B, S, E), jnp.float32).reshape(T, E)
    BH = 1536
    NH = H // BH
    grid = (E + NS, NH)

    def we_map(e, h):
        return (jnp.minimum(e, E - 1), 0, jnp.where(e < E, h, NH - 1))

    def wed_map(e, h):
        return (jnp.minimum(e, E - 1), jnp.where(e < E, h, NH - 1), 0)

    def ws_map(e, h):
        return (jnp.maximum(e - E, 0), 0, jnp.where(e < E, 0, h))

    def wsd_map(e, h):
        return (jnp.maximum(e - E, 0), jnp.where(e < E, 0, h), 0)

    const2 = lambda e, h: (0, 0)
    body = functools.partial(_moe_body, E=E, NH=NH)
    out = pl.pallas_call(
        body,
        grid=grid,
        in_specs=[
            pl.BlockSpec((T, D), const2),
            pl.BlockSpec((D, E), const2),
            pl.BlockSpec((1, E), const2),
            pl.BlockSpec((D, E), const2),
            pl.BlockSpec((1, E), const2),
            pl.BlockSpec((T, E), const2),
            pl.BlockSpec((1, D, BH), we_map),
            pl.BlockSpec((1, D, BH), we_map),
            pl.BlockSpec((1, BH, D), wed_map),
            pl.BlockSpec((1, D, BH), ws_map),
            pl.BlockSpec((1, D, BH), ws_map),
            pl.BlockSpec((1, BH, D), wsd_map),
        ],
        out_specs=pl.BlockSpec((T, D), const2),
        out_shape=jax.ShapeDtypeStruct((T, D), jnp.float32),
        scratch_shapes=[
            pltpu.VMEM((T, E), jnp.float32),
            pltpu.VMEM((E, T), jnp.float32),
            pltpu.VMEM((E, T), jnp.float32),
            pltpu.VMEM((T, D), jnp.float32),
            pltpu.VMEM((T, D), jnp.float32),
        ],
        compiler_params=pltpu.CompilerParams(
            dimension_semantics=("arbitrary", "arbitrary"),
            vmem_limit_bytes=128 * 1024 * 1024),
    )(xf, Wg, bg.reshape(1, E), Wn, bn.reshape(1, E), noise,
      We_gate, We_up, We_down, Ws_gate, Ws_up, Ws_down)
    return out.reshape(B, S, D)
